# Initial kernel scaffold; baseline (speedup 1.0000x reference)
import jax
import jax.numpy as jnp
from jax.experimental import pallas as pl


def kernel(x, pos, edge_index, batch, params):
    raise NotImplementedError("write your pallas kernel here")


# speedup vs baseline: 14.4673x; 14.4673x over previous
import functools

import jax
import jax.numpy as jnp
from jax import lax
from jax.experimental import pallas as pl
from jax.experimental.pallas import tpu as pltpu
from jax.experimental.pallas import tpu_sc as plsc

F = 128
MD = 16
N = 10000
E = 320000
G = 128
E8 = E // 8

NC, NS = 2, 16
NW = NC * NS
EW = E // NW
CH = 1000
GW = 125
GSUB = CH // GW
NCHUNK = EW // CH

_MESH = plsc.VectorSubcoreMesh(core_axis_name="c", subcore_axis_name="s")


def _sc_pair(width: int, chunk: int):
    gsub = chunk // GW
    nchunk = EW // chunk
    @functools.partial(
        pl.kernel,
        out_type=jax.ShapeDtypeStruct((E, width), jnp.float32),
        mesh=_MESH,
        compiler_params=pltpu.CompilerParams(use_tc_tiling_on_sc=False),
        scratch_types=[
            pltpu.VMEM((2, gsub, GW), jnp.int32),
            pltpu.VMEM((2, gsub, GW), jnp.int32),
            pltpu.VMEM((2, chunk, width), jnp.float32),
            pltpu.VMEM((2, chunk, width), jnp.float32),
            pltpu.VMEM((chunk, width), jnp.float32),
            pltpu.SemaphoreType.DMA,
            pltpu.SemaphoreType.DMA,
        ],
    )
    def k(td_hbm, ts_hbm, didx_hbm, sidx_hbm, out_hbm,
          didx, sidx, drows, srows, orows, sem_i, sem_g):
        wid = lax.axis_index("s") * NC + lax.axis_index("c")
        c0 = wid * nchunk

        def fire_idx(ci, b):
            return [pltpu.async_copy(didx_hbm.at[c0 + ci], didx.at[b], sem_i),
                    pltpu.async_copy(sidx_hbm.at[c0 + ci], sidx.at[b], sem_i)]

        def fire_gathers(b):
            descs = []
            for j in range(gsub):
                descs.append(pltpu.async_copy(
                    td_hbm.at[didx.at[b, j]],
                    drows.at[b, pl.ds(j * GW, GW)], sem_g))
                descs.append(pltpu.async_copy(
                    ts_hbm.at[sidx.at[b, j]],
                    srows.at[b, pl.ds(j * GW, GW)], sem_g))
            return descs

        for d in fire_idx(0, 0):
            d.wait()
        g = [fire_gathers(0), None]
        ip = [None, fire_idx(1, 1) if nchunk > 1 else None]

        for ci in range(nchunk):
            b = ci % 2
            if ci + 1 < nchunk:
                for d in ip[1 - b]:
                    d.wait()
                g[1 - b] = fire_gathers(1 - b)
            for d in g[b]:
                d.wait()
            if ci + 2 < nchunk:
                ip[b] = fire_idx(ci + 2, b)

            def body(i, c2):
                for w in range(0, width, MD):
                    orows[i, pl.ds(w, MD)] = (drows[b, i, pl.ds(w, MD)]
                                              + srows[b, i, pl.ds(w, MD)])
                return c2
            lax.fori_loop(0, chunk, body, 0, unroll=8)
            pltpu.sync_copy(orows,
                            out_hbm.at[pl.ds(wid * EW + ci * chunk, chunk)])

    return k


_sc_pair_add = _sc_pair(MD, CH)

CHQ = 500
GSUBQ = CHQ // GW
NCHUNKQ = EW // CHQ


@functools.partial(
    pl.kernel,
    out_type=(jax.ShapeDtypeStruct((E, MD), jnp.float32),
              jax.ShapeDtypeStruct((E, MD), jnp.float32)),
    mesh=_MESH,
    compiler_params=pltpu.CompilerParams(use_tc_tiling_on_sc=False),
    scratch_types=[
        pltpu.VMEM((2, GSUBQ, GW), jnp.int32),
        pltpu.VMEM((2, GSUBQ, GW), jnp.int32),
        pltpu.VMEM((2, CHQ, MD), jnp.float32),
        pltpu.VMEM((2, CHQ, MD), jnp.float32),
        pltpu.VMEM((2, CHQ, MD), jnp.float32),
        pltpu.VMEM((2, CHQ, MD), jnp.float32),
        pltpu.VMEM((CHQ, MD), jnp.float32),
        pltpu.VMEM((CHQ, MD), jnp.float32),
        pltpu.SemaphoreType.DMA,
        pltpu.SemaphoreType.DMA,
    ],
)
def _sc_pair_quad(pp_hbm, a_hbm, b_hbm, didx_hbm, sidx_hbm,
                  rel_hbm, s_hbm,
                  didx, sidx, pdr, psr, arows, brows, orel, os0,
                  sem_i, sem_g):
    wid = lax.axis_index("s") * NC + lax.axis_index("c")
    c0 = wid * NCHUNKQ

    def fire_idx(ci, b):
        return [pltpu.async_copy(didx_hbm.at[c0 + ci], didx.at[b], sem_i),
                pltpu.async_copy(sidx_hbm.at[c0 + ci], sidx.at[b], sem_i)]

    def fire_gathers(b):
        descs = []
        for j in range(GSUBQ):
            sl = pl.ds(j * GW, GW)
            descs.append(pltpu.async_copy(
                pp_hbm.at[didx.at[b, j]], pdr.at[b, sl], sem_g))
            descs.append(pltpu.async_copy(
                pp_hbm.at[sidx.at[b, j]], psr.at[b, sl], sem_g))
            descs.append(pltpu.async_copy(
                a_hbm.at[didx.at[b, j]], arows.at[b, sl], sem_g))
            descs.append(pltpu.async_copy(
                b_hbm.at[sidx.at[b, j]], brows.at[b, sl], sem_g))
        return descs

    for d in fire_idx(0, 0):
        d.wait()
    g = [fire_gathers(0), None]
    ip = [None, fire_idx(1, 1)]

    for ci in range(NCHUNKQ):
        b = ci % 2
        if ci + 1 < NCHUNKQ:
            for d in ip[1 - b]:
                d.wait()
            g[1 - b] = fire_gathers(1 - b)
        for d in g[b]:
            d.wait()
        if ci + 2 < NCHUNKQ:
            ip[b] = fire_idx(ci + 2, b)

        def body(i, c2):
            orel[i, :] = psr[b, i, :] - pdr[b, i, :]
            os0[i, :] = arows[b, i, :] + brows[b, i, :]
            return c2
        lax.fori_loop(0, CHQ, body, 0, unroll=8)
        base = wid * EW + ci * CHQ
        pltpu.sync_copy(orel, rel_hbm.at[pl.ds(base, CHQ)])
        pltpu.sync_copy(os0, s_hbm.at[pl.ds(base, CHQ)])

    return None

N_ACC = 10240
_RPT = N_ACC // NS


@functools.partial(
    pl.kernel,
    out_type=jax.ShapeDtypeStruct((NC, N_ACC, MD), jnp.float32),
    mesh=_MESH,
    compiler_params=pltpu.CompilerParams(use_tc_tiling_on_sc=False),
    scratch_types=[
        pltpu.VMEM((3, GSUB, GW), jnp.int32),
        pltpu.VMEM((3, CH, MD), jnp.float32),
        pltpu.VMEM((_RPT, MD), jnp.float32),
        pltpu.VMEM_SHARED((N_ACC, MD), jnp.float32),
        pltpu.SemaphoreType.DMA,
        pltpu.SemaphoreType.DMA,
    ],
)
def _sc_segsum(m_hbm, didx_hbm, out_hbm, didx, mrows, rbuf, acc,
               sem_l, sem_s):
    cid = lax.axis_index("c")
    sid = lax.axis_index("s")
    wid = sid * NC + cid
    c0 = wid * NCHUNK

    def fire_loads(ci, b):
        return [pltpu.async_copy(didx_hbm.at[c0 + ci], didx.at[b], sem_l),
                pltpu.async_copy(m_hbm.at[pl.ds(wid * EW + ci * CH, CH)],
                                 mrows.at[b], sem_l)]

    def fire_scatters(b):
        return [pltpu.async_copy(mrows.at[b, pl.ds(j * GW, GW)],
                                 acc.at[didx.at[b, j]], sem_s, add=True)
                for j in range(GSUB)]

    ld = [fire_loads(0, 0), fire_loads(1, 1) if NCHUNK > 1 else None, None]

    def z(i, c2):
        rbuf[i, :] = jnp.zeros((MD,), jnp.float32)
        return c2
    lax.fori_loop(0, _RPT, z, 0, unroll=8)
    pltpu.sync_copy(rbuf, acc.at[pl.ds(sid * _RPT, _RPT)])
    plsc.subcore_barrier()

    sc = [None, None, None]
    for ci in range(NCHUNK):
        b = ci % 3
        for d in ld[b]:
            d.wait()
        sc[b] = fire_scatters(b)
        b2 = (ci - 1) % 3
        if sc[b2] is not None and ci >= 1:
            for d in sc[b2]:
                d.wait()
            sc[b2] = None
        if ci + 2 < NCHUNK:
            ld[b2] = fire_loads(ci + 2, b2)
    for s in sc:
        if s is not None:
            for d in s:
                d.wait()

    plsc.subcore_barrier()
    pltpu.sync_copy(acc.at[pl.ds(sid * _RPT, _RPT)], rbuf)
    pltpu.sync_copy(rbuf, out_hbm.at[cid, pl.ds(sid * _RPT, _RPT)])


N_PAD = 10240
_PW = N_PAD // NW
_PG = _PW // 80


@functools.partial(
    pl.kernel,
    out_type=jax.ShapeDtypeStruct((NC, G, 32), jnp.float32),
    mesh=_MESH,
    compiler_params=pltpu.CompilerParams(use_tc_tiling_on_sc=False),
    scratch_types=[
        pltpu.VMEM((_PG, 80), jnp.int32),
        pltpu.VMEM((_PW, 32), jnp.float32),
        pltpu.VMEM((G // NS, 32), jnp.float32),
        pltpu.VMEM_SHARED((G, 32), jnp.float32),
    ],
)
def _sc_pool(s_hbm, bidx_hbm, out_hbm, bidx, sbuf, rbuf, acc):
    cid = lax.axis_index("c")
    sid = lax.axis_index("s")
    wid = sid * NC + cid
    gpt = G // NS

    for i in range(gpt):
        rbuf[i, pl.ds(0, 16)] = jnp.zeros((16,), jnp.float32)
        rbuf[i, pl.ds(16, 16)] = jnp.zeros((16,), jnp.float32)
    pltpu.sync_copy(rbuf, acc.at[pl.ds(sid * gpt, gpt)])
    plsc.subcore_barrier()

    pltpu.sync_copy(bidx_hbm.at[wid], bidx)
    pltpu.sync_copy(s_hbm.at[pl.ds(wid * _PW, _PW)], sbuf)
    for j in range(_PG):
        pltpu.sync_copy(sbuf.at[pl.ds(j * 80, 80)],
                        acc.at[bidx.at[j]], add=True)
    plsc.subcore_barrier()
    pltpu.sync_copy(acc.at[pl.ds(sid * gpt, gpt)], rbuf)
    pltpu.sync_copy(rbuf, out_hbm.at[cid, pl.ds(sid * gpt, gpt)])


_BKN = 1000
_BKE = 2000


def _full(shape):
    return pl.BlockSpec(shape, lambda i: (0, 0))


def _tc_pre_body(x_ref, wi_ref, wj_ref, a_ref, b_ref):
    xb = x_ref[...]
    a_ref[...] = jnp.dot(xb, wi_ref[...], preferred_element_type=jnp.float32)
    b_ref[...] = jnp.dot(xb, wj_ref[...], preferred_element_type=jnp.float32)


_tc_pre = pl.pallas_call(
    _tc_pre_body,
    grid=(N // _BKN,),
    in_specs=[
        pl.BlockSpec((_BKN, F), lambda i: (i, 0)),
        _full((F, MD)),
        _full((F, MD)),
    ],
    out_specs=[
        pl.BlockSpec((_BKN, MD), lambda i: (i, 0)),
        pl.BlockSpec((_BKN, MD), lambda i: (i, 0)),
    ],
    out_shape=[
        jax.ShapeDtypeStruct((N, MD), jnp.float32),
        jax.ShapeDtypeStruct((N, MD), jnp.float32),
    ],
)


def _tc_mid0_body(s_ref, rel_ref, sel16_ref, sel8_ref, w1dt_ref, b1t_ref,
                  w2bd_ref, b2t_ref, m_ref, rd_ref):
    rel = rel_ref[...]
    rd8 = jnp.sqrt(jnp.dot(rel * rel, sel16_ref[...],
                           preferred_element_type=jnp.float32))
    rd128 = jnp.dot(rd8, sel8_ref[...], preferred_element_type=jnp.float32)
    t = s_ref[...] + rd128 * w1dt_ref[...] + b1t_ref[...]
    m = jax.nn.silu(t)
    m2 = jnp.dot(m, w2bd_ref[...], preferred_element_type=jnp.float32) \
        + b2t_ref[...]
    m_ref[...] = jax.nn.silu(m2)
    rd_ref[...] = rd8


_tc_mid0 = pl.pallas_call(
    _tc_mid0_body,
    grid=(E8 // _BKE,),
    in_specs=[
        pl.BlockSpec((_BKE, F), lambda i: (i, 0)),
        pl.BlockSpec((_BKE, F), lambda i: (i, 0)),
        _full((F, 8)),
        _full((8, F)),
        _full((1, F)),
        _full((1, F)),
        _full((F, F)),
        _full((1, F)),
    ],
    out_specs=[
        pl.BlockSpec((_BKE, F), lambda i: (i, 0)),
        pl.BlockSpec((_BKE, 8), lambda i: (i, 0)),
    ],
    out_shape=[
        jax.ShapeDtypeStruct((E8, F), jnp.float32),
        jax.ShapeDtypeStruct((E8, 8), jnp.float32),
    ],
)


def _tc_mid_body(s_ref, rd_ref, sel_ref, w1dt_ref, b1t_ref, w2bd_ref,
                 b2t_ref, m_ref):
    rd128 = jnp.dot(rd_ref[...], sel_ref[...],
                    preferred_element_type=jnp.float32)
    t = s_ref[...] + rd128 * w1dt_ref[...] + b1t_ref[...]
    m = jax.nn.silu(t)
    m2 = jnp.dot(m, w2bd_ref[...], preferred_element_type=jnp.float32) \
        + b2t_ref[...]
    m_ref[...] = jax.nn.silu(m2)


_tc_mid = pl.pallas_call(
    _tc_mid_body,
    grid=(E8 // _BKE,),
    in_specs=[
        pl.BlockSpec((_BKE, F), lambda i: (i, 0)),
        pl.BlockSpec((_BKE, 8), lambda i: (i, 0)),
        _full((8, F)),
        _full((1, F)),
        _full((1, F)),
        _full((F, F)),
        _full((1, F)),
    ],
    out_specs=pl.BlockSpec((_BKE, F), lambda i: (i, 0)),
    out_shape=jax.ShapeDtypeStruct((E8, F), jnp.float32),
)


def _tc_node_body(out_ref, mia_ref, mib_ref, wn1x_ref, wn1m_ref, bn1_ref,
                  wn2_ref, bn2_ref, wi_ref, wj_ref,
                  out2_ref, a_ref, b_ref):
    o = out_ref[...]
    mi = mia_ref[0] + mib_ref[0]
    t = jax.nn.silu(
        jnp.dot(o, wn1x_ref[...], preferred_element_type=jnp.float32)
        + jnp.dot(mi, wn1m_ref[...], preferred_element_type=jnp.float32)
        + bn1_ref[...])
    o2 = jnp.dot(t, wn2_ref[...], preferred_element_type=jnp.float32) \
        + bn2_ref[...] + o
    out2_ref[...] = o2
    a_ref[...] = jnp.dot(o2, wi_ref[...], preferred_element_type=jnp.float32)
    b_ref[...] = jnp.dot(o2, wj_ref[...], preferred_element_type=jnp.float32)


_tc_node = pl.pallas_call(
    _tc_node_body,
    grid=(N // _BKN,),
    in_specs=[
        pl.BlockSpec((_BKN, F), lambda i: (i, 0)),
        pl.BlockSpec((1, _BKN, MD), lambda i: (0, i, 0)),
        pl.BlockSpec((1, _BKN, MD), lambda i: (1, i, 0)),
        _full((F, MD)),
        _full((MD, MD)),
        _full((1, MD)),
        _full((MD, F)),
        _full((1, F)),
        _full((F, MD)),
        _full((F, MD)),
    ],
    out_specs=[
        pl.BlockSpec((_BKN, F), lambda i: (i, 0)),
        pl.BlockSpec((_BKN, MD), lambda i: (i, 0)),
        pl.BlockSpec((_BKN, MD), lambda i: (i, 0)),
    ],
    out_shape=[
        jax.ShapeDtypeStruct((N, F), jnp.float32),
        jax.ShapeDtypeStruct((N, MD), jnp.float32),
        jax.ShapeDtypeStruct((N, MD), jnp.float32),
    ],
)


def _tc_node_last_body(out_ref, mia_ref, mib_ref, wn1x_ref, wn1m_ref,
                       bn1_ref, wn2_ref, bn2_ref, wl1_ref, bl1_ref,
                       emb_ref, oneh_ref, saug_ref):
    o = out_ref[...]
    mi = mia_ref[0] + mib_ref[0]
    t = jax.nn.silu(
        jnp.dot(o, wn1x_ref[...], preferred_element_type=jnp.float32)
        + jnp.dot(mi, wn1m_ref[...], preferred_element_type=jnp.float32)
        + bn1_ref[...])
    o2 = jnp.dot(t, wn2_ref[...], preferred_element_type=jnp.float32) \
        + bn2_ref[...] + o
    s = jax.nn.silu(
        jnp.dot(o2, wl1_ref[...], preferred_element_type=jnp.float32)
        + bl1_ref[...])
    saug_ref[...] = jnp.dot(s, emb_ref[...],
                            preferred_element_type=jnp.float32) \
        + oneh_ref[...]


_tc_node_last = pl.pallas_call(
    _tc_node_last_body,
    grid=(N // _BKN,),
    in_specs=[
        pl.BlockSpec((_BKN, F), lambda i: (i, 0)),
        pl.BlockSpec((1, _BKN, MD), lambda i: (0, i, 0)),
        pl.BlockSpec((1, _BKN, MD), lambda i: (1, i, 0)),
        _full((F, MD)),
        _full((MD, MD)),
        _full((1, MD)),
        _full((MD, F)),
        _full((1, F)),
        _full((F, MD)),
        _full((1, MD)),
        _full((MD, 32)),
        _full((1, 32)),
    ],
    out_specs=pl.BlockSpec((_BKN, 32), lambda i: (i, 0)),
    out_shape=jax.ShapeDtypeStruct((N, 32), jnp.float32),
)


def _tc_final_body(p_ref, wl2_ref, bl2_ref, wl3_ref, bl3_ref, wl4_ref,
                   bl4_ref, o_ref):
    p = p_ref[0] + p_ref[1]
    ps = p[:, :MD]
    cnt = p[:, MD:MD + 1]
    pooled = jnp.dot(ps, wl2_ref[...], preferred_element_type=jnp.float32) \
        + cnt * bl2_ref[...]
    h = jax.nn.silu(
        jnp.dot(pooled, wl3_ref[...], preferred_element_type=jnp.float32)
        + bl3_ref[...])
    o_ref[...] = jnp.dot(h, wl4_ref[...], preferred_element_type=jnp.float32) \
        + bl4_ref[...]


_tc_final = pl.pallas_call(
    _tc_final_body,
    in_specs=[
        pl.BlockSpec((NC, G, 32), lambda: (0, 0, 0)),
        pl.BlockSpec((MD, F), lambda: (0, 0)),
        pl.BlockSpec((1, F), lambda: (0, 0)),
        pl.BlockSpec((F, MD), lambda: (0, 0)),
        pl.BlockSpec((1, MD), lambda: (0, 0)),
        pl.BlockSpec((MD, 1), lambda: (0, 0)),
        pl.BlockSpec((1, 1), lambda: (0, 0)),
    ],
    out_specs=pl.BlockSpec((G, 1), lambda: (0, 0)),
    out_shape=jax.ShapeDtypeStruct((G, 1), jnp.float32),
)


def kernel(x, pos, edge_index, batch, params):
    src = edge_index[0]
    dst = edge_index[1]
    didx3 = dst.reshape(E // CH, GSUB, GW)
    sidx3 = src.reshape(E // CH, GSUB, GW)
    didx5 = dst.reshape(E // CHQ, GSUBQ, GW)
    sidx5 = src.reshape(E // CHQ, GSUBQ, GW)
    posp = jnp.pad(pos, ((0, 0), (0, MD - 3)))

    sel16 = (jnp.arange(F)[:, None] // MD == jnp.arange(8)[None, :]
             ).astype(jnp.float32)
    sel8 = sel16.T

    lps = params["layers"]
    wi = [lp["e1"]["w"][:F] for lp in lps]
    wj = [lp["e1"]["w"][F:2 * F] for lp in lps]

    out = x
    a, b = _tc_pre(x, wi[0], wj[0])
    rd8 = None
    for l in range(3):
        lp = lps[l]
        w1d = lp["e1"]["w"][2 * F]
        b1 = lp["e1"]["b"]
        w2 = lp["e2"]["w"]
        b2 = lp["e2"]["b"]
        w1dt = jnp.tile(w1d, 8)[None, :]
        b1t = jnp.tile(b1, 8)[None, :]
        w2bd = jnp.kron(jnp.eye(8, dtype=jnp.float32), w2)
        b2t = jnp.tile(b2, 8)[None, :]
        if l == 0:
            rel, s0 = _sc_pair_quad(posp, a, b, didx5, sidx5)
            m8, rd8 = _tc_mid0(s0.reshape(E8, F), rel.reshape(E8, F),
                               sel16, sel8, w1dt, b1t, w2bd, b2t)
        else:
            s = _sc_pair_add(a, b, didx3, sidx3)
            m8 = _tc_mid(s.reshape(E8, F), rd8, sel8,
                         w1dt, b1t, w2bd, b2t)
        mm = m8.reshape(E, MD)
        mip = _sc_segsum(mm, didx3)
        wn1 = lp["n1"]["w"]
        if l < 2:
            out, a, b = _tc_node(
                out, mip, mip, wn1[:F], wn1[F:], lp["n1"]["b"][None, :],
                lp["n2"]["w"], lp["n2"]["b"][None, :], wi[l + 1], wj[l + 1])
        else:
            emb = jnp.concatenate(
                [jnp.eye(MD, dtype=jnp.float32),
                 jnp.zeros((MD, 16), jnp.float32)], axis=1)
            oneh = jnp.zeros((1, 32), jnp.float32).at[0, MD].set(1.0)
            saug = _tc_node_last(
                out, mip, mip, wn1[:F], wn1[F:], lp["n1"]["b"][None, :],
                lp["n2"]["w"], lp["n2"]["b"][None, :],
                params["lin1"]["w"], params["lin1"]["b"][None, :],
                emb, oneh)

    saug_p = jnp.pad(saug, ((0, N_PAD - N), (0, 0)))
    batch_p = jnp.pad(batch, (0, N_PAD - N)).reshape(NW, _PG, 80)
    paug = _sc_pool(saug_p, batch_p)

    o = _tc_final(paug, params["lin2"]["w"], params["lin2"]["b"][None, :],
                  params["lin3"]["w"], params["lin3"]["b"][None, :],
                  params["lin4"]["w"],
                  params["lin4"]["b"][None, :])
    return o.reshape(-1)
